# Initial kernel scaffold; baseline (speedup 1.0000x reference)
import jax
import jax.numpy as jnp
from jax.experimental import pallas as pl


def kernel(output, target):
    raise NotImplementedError("write your pallas kernel here")


# speedup vs baseline: 30.7720x; 30.7720x over previous
import jax
import jax.numpy as jnp
from jax.experimental import pallas as pl
from jax.experimental.pallas import tpu as pltpu

_N = None


def _trimmed_mean_body(o_ref, t_ref, out_ref):
    o = o_ref[...]
    t = t_ref[...]
    r = (t - o) / t
    e = r * r
    ebits = jax.lax.bitcast_convert_type(e, jnp.int32)
    n = e.size
    k = int(n * 0.97)

    def step(i, u):
        bit = jnp.int32(30) - i
        cand = u | jnp.left_shift(jnp.int32(1), bit)
        cnt = jnp.sum((ebits < cand).astype(jnp.int32))
        return jnp.where(cnt < k, cand, u)

    u = jax.lax.fori_loop(0, 31, step, jnp.int32(0))
    tval = jax.lax.bitcast_convert_type(u, jnp.float32)
    lt = ebits < u
    s = jnp.sum(jnp.where(lt, e, jnp.float32(0.0)))
    c = jnp.sum(lt.astype(jnp.int32))
    out_ref[0, 0] = (s + (k - c).astype(jnp.float32) * tval) / jnp.float32(k)


def kernel(output, target):
    n = output.shape[0]
    o2 = output.reshape(n // 128, 128)
    t2 = target.reshape(n // 128, 128)
    res = pl.pallas_call(
        _trimmed_mean_body,
        out_shape=jax.ShapeDtypeStruct((1, 1), jnp.float32),
        out_specs=pl.BlockSpec(memory_space=pltpu.SMEM),
    )(o2, t2)
    return res.reshape(())
